# baseline (speedup 1.0000x reference)
import functools
import numpy as np
import jax
import jax.numpy as jnp
from jax import lax
from jax.experimental import pallas as pl
from jax.experimental.pallas import tpu as pltpu
from jax.experimental.pallas import tpu_sc as plsc

_ELT = 64
_ROW = 65536
_T2 = np.float32((2.0 * np.sin(np.radians(10.0) / 2.0)) ** 2)
_NC, _NS, _L = 2, 16, 16
_NW = _NC * _NS
_SEG_PER_W = _ELT // _NW
_P = 8192
_NCHUNK = _ROW // _P
_UNROLL = 4

_LN2 = np.float32(0.6931471805599453)
_SQRT2 = np.float32(1.4142135623730951)


def _softlog(p):
    bits = lax.bitcast_convert_type(p, jnp.int32)
    e = jnp.right_shift(bits, 23) - 127
    m = lax.bitcast_convert_type((bits & 0x007FFFFF) | 0x3F800000, jnp.float32)
    big = m > _SQRT2
    m = jnp.where(big, m * np.float32(0.5), m)
    ef = (e + jnp.where(big, 1, 0)).astype(jnp.float32)
    f = m - np.float32(1.0)
    t = f / (np.float32(2.0) + f)
    t2 = t * t
    poly = np.float32(2.0) + t2 * (
        np.float32(2.0 / 3.0) + t2 * (
            np.float32(0.4) + t2 * (
                np.float32(2.0 / 7.0) + t2 * np.float32(2.0 / 9.0))))
    return t * poly + ef * _LN2


_N = _ELT * _ROW


def _sc_body(pt, ot, hb, ceb, out_hbm, buf, hv, cev, outv):
    wid = lax.axis_index("s") * _NC + lax.axis_index("c")
    sums = []
    for j in range(_SEG_PER_W):
        seg = wid * _SEG_PER_W + j
        pltpu.sync_copy(hb.at[pl.ds(seg * _L, _L)], hv)
        pltpu.sync_copy(ceb.at[pl.ds(seg * _L, _L)], cev)
        hvec = hv[...]
        cevec = cev[...]
        omh = np.float32(1.0) - hvec
        acc = jnp.zeros((_L,), jnp.float32)
        for k in range(_NCHUNK):
            base = seg * _ROW + k * _P
            for c in range(3):
                pltpu.sync_copy(pt.at[pl.ds(c * _N + base, _P)], buf.at[c])
                pltpu.sync_copy(ot.at[pl.ds(c * _N + base, _P)], buf.at[3 + c])

            def inner(i, acc):
                for u in range(_UNROLL):
                    off = i * (_L * _UNROLL) + u * _L
                    dx = buf[0, pl.ds(off, _L)] - buf[3, pl.ds(off, _L)]
                    dy = buf[1, pl.ds(off, _L)] - buf[4, pl.ds(off, _L)]
                    dz = buf[2, pl.ds(off, _L)] - buf[5, pl.ds(off, _L)]
                    s2 = dx * dx + dy * dy + dz * dz
                    pe = hvec * jnp.exp(s2 * cevec) + omh
                    pe = jnp.where(s2 < _T2, pe, np.float32(1.0))
                    acc = acc + _softlog(pe)
                return acc

            acc = lax.fori_loop(0, _P // (_L * _UNROLL), inner, acc)
        outv[...] = acc
        pltpu.sync_copy(outv, out_hbm.at[pl.ds(seg * _L, _L)])


@jax.jit
def kernel(u_pred, u_obs, h, lam):
    pt = u_pred.T.reshape(3 * _N)
    ot = u_obs.T.reshape(3 * _N)
    hb = jnp.broadcast_to(h[:, None], (_ELT, _L)).reshape(_ELT * _L)
    ceb = jnp.broadcast_to((lam * (-1.0 / _T2))[:, None],
                           (_ELT, _L)).reshape(_ELT * _L)
    mesh = plsc.VectorSubcoreMesh(core_axis_name="c", subcore_axis_name="s")
    out = pl.kernel(
        _sc_body,
        mesh=mesh,
        out_type=jax.ShapeDtypeStruct((_ELT * _L,), jnp.float32),
        scratch_types=[
            pltpu.VMEM((6, _P), jnp.float32),
            pltpu.VMEM((_L,), jnp.float32),
            pltpu.VMEM((_L,), jnp.float32),
            pltpu.VMEM((_L,), jnp.float32),
        ],
    )(pt, ot, hb, ceb)
    return out.reshape(_ELT, _L).sum(axis=1)


# speedup vs baseline: 3.8508x; 3.8508x over previous
import functools
import numpy as np
import jax
import jax.numpy as jnp
from jax import lax
from jax.experimental import pallas as pl
from jax.experimental.pallas import tpu as pltpu
from jax.experimental.pallas import tpu_sc as plsc

_ELT = 64
_ROW = 65536
_T2 = np.float32((2.0 * np.sin(np.radians(10.0) / 2.0)) ** 2)
_NC, _NS, _L = 2, 16, 16
_NW = _NC * _NS
_SEG_PER_W = _ELT // _NW
_NROWS = _ELT * _ROW // 128
_ROWSEG = _ROW // 128
_PR = 64
_NCHUNK = _ROWSEG // _PR

_LN2 = np.float32(0.6931471805599453)
_SQRT2 = np.float32(1.4142135623730951)


def _softlog(p):
    bits = lax.bitcast_convert_type(p, jnp.int32)
    e = jnp.right_shift(bits, 23) - 127
    m = lax.bitcast_convert_type((bits & 0x007FFFFF) | 0x3F800000, jnp.float32)
    big = m > _SQRT2
    m = jnp.where(big, m * np.float32(0.5), m)
    ef = (e + jnp.where(big, 1, 0)).astype(jnp.float32)
    f = m - np.float32(1.0)
    t = f / (np.float32(2.0) + f)
    t2 = t * t
    poly = np.float32(2.0) + t2 * (
        np.float32(2.0 / 3.0) + t2 * (
            np.float32(0.4) + t2 * (
                np.float32(2.0 / 7.0) + t2 * np.float32(2.0 / 9.0))))
    return t * poly + ef * _LN2


def _sc_body(pt, ot, hb, ceb, out_hbm, buf, hv, cev, outv):
    wid = lax.axis_index("s") * _NC + lax.axis_index("c")
    for j in range(_SEG_PER_W):
        seg = wid * _SEG_PER_W + j
        pltpu.sync_copy(hb.at[pl.ds(seg * _L, _L)], hv)
        pltpu.sync_copy(ceb.at[pl.ds(seg * _L, _L)], cev)
        hvec = hv[...]
        cevec = cev[...]
        omh = np.float32(1.0) - hvec
        seg_row = seg * _ROWSEG

        def chunk_body(k, acc):
            row0 = seg_row + k * _PR
            for c in range(3):
                pltpu.sync_copy(pt.at[pl.ds(c * _NROWS + row0, _PR)],
                                buf.at[c])
                pltpu.sync_copy(ot.at[pl.ds(c * _NROWS + row0, _PR)],
                                buf.at[3 + c])

            def inner(r, acc):
                for u in range(8):
                    sl = pl.ds(u * _L, _L)
                    dx = buf[0, r, sl] - buf[3, r, sl]
                    dy = buf[1, r, sl] - buf[4, r, sl]
                    dz = buf[2, r, sl] - buf[5, r, sl]
                    s2 = dx * dx + dy * dy + dz * dz
                    pe = hvec * jnp.exp(s2 * cevec) + omh
                    pe = jnp.where(s2 < _T2, pe, np.float32(1.0))
                    acc = acc + _softlog(pe)
                return acc

            return lax.fori_loop(0, _PR, inner, acc)

        acc = lax.fori_loop(0, _NCHUNK, chunk_body,
                            jnp.zeros((_L,), jnp.float32))
        outv[...] = acc
        pltpu.sync_copy(outv, out_hbm.at[pl.ds(seg * _L, _L)])


@jax.jit
def kernel(u_pred, u_obs, h, lam):
    pt = u_pred.T.reshape(3 * _NROWS, 128)
    ot = u_obs.T.reshape(3 * _NROWS, 128)
    hb = jnp.broadcast_to(h[:, None], (_ELT, _L)).reshape(_ELT * _L)
    ceb = jnp.broadcast_to((lam * (-1.0 / _T2))[:, None],
                           (_ELT, _L)).reshape(_ELT * _L)
    mesh = plsc.VectorSubcoreMesh(core_axis_name="c", subcore_axis_name="s")
    out = pl.kernel(
        _sc_body,
        mesh=mesh,
        out_type=jax.ShapeDtypeStruct((_ELT * _L,), jnp.float32),
        scratch_types=[
            pltpu.VMEM((6, _PR, 128), jnp.float32),
            pltpu.VMEM((_L,), jnp.float32),
            pltpu.VMEM((_L,), jnp.float32),
            pltpu.VMEM((_L,), jnp.float32),
        ],
    )(pt, ot, hb, ceb)
    return out.reshape(_ELT, _L).sum(axis=1)


# speedup vs baseline: 12.7995x; 3.3239x over previous
import functools
import numpy as np
import jax
import jax.numpy as jnp
from jax.experimental import pallas as pl
from jax.experimental.pallas import tpu as pltpu

_ELT = 64
_ROW = 65536
_C = 1024
_R = _ELT * _ROW // _C
_RSEG = _ROW // _C
_T2 = np.float32((2.0 * np.sin(np.radians(10.0) / 2.0)) ** 2)


def _tc_body(p_ref, o_ref, h_ref, lam_ref, out_ref):
    d = p_ref[...] - o_ref[...]
    d2 = d * d
    s2 = d2[0] + d2[1] + d2[2]
    h = h_ref[0, 0, 0]
    lam = lam_ref[0, 0, 0]
    p = h * jnp.exp(s2 * (-1.0 / _T2) * lam) + (1.0 - h)
    log_p = jnp.where(s2 < _T2, jnp.log(p), 0.0)
    out_ref[...] = jnp.sum(log_p, dtype=jnp.float32)[None, None, None] * jnp.ones(
        (1, 1, 128), jnp.float32)


@jax.jit
def kernel(u_pred, u_obs, h, lam):
    pt = u_pred.T.reshape(3, _R, _C)
    ot = u_obs.T.reshape(3, _R, _C)
    hb = jnp.broadcast_to(h[:, None, None], (_ELT, 1, 128))
    lb = jnp.broadcast_to(lam[:, None, None], (_ELT, 1, 128))
    out = pl.pallas_call(
        _tc_body,
        grid=(_ELT,),
        in_specs=[
            pl.BlockSpec((3, _RSEG, _C), lambda e: (0, e, 0)),
            pl.BlockSpec((3, _RSEG, _C), lambda e: (0, e, 0)),
            pl.BlockSpec((1, 1, 128), lambda e: (e, 0, 0)),
            pl.BlockSpec((1, 1, 128), lambda e: (e, 0, 0)),
        ],
        out_specs=pl.BlockSpec((1, 1, 128), lambda e: (e, 0, 0)),
        out_shape=jax.ShapeDtypeStruct((_ELT, 1, 128), jnp.float32),
    )(pt, ot, hb, lb)
    return out[:, 0, 0]
